# baseline (speedup 1.0000x reference)
import functools

import jax
import jax.numpy as jnp
from jax import lax
from jax.experimental import pallas as pl
from jax.experimental.pallas import tpu as pltpu
from jax.experimental.pallas import tpu_sc as plsc

F32 = jnp.float32

_NC, _NS = 2, 16
_NW = _NC * _NS
_IDXW = 128
_CB = 2
_TW = 128


def _sc_gather(table, idx2):
    groups = idx2.shape[0]
    nrows = table.shape[0]
    per_w_groups = groups // _NW
    iters = per_w_groups // 2
    rows_per_sub = nrows // _NS
    mesh = plsc.VectorSubcoreMesh(core_axis_name="c", subcore_axis_name="s")

    @functools.partial(
        pl.kernel,
        mesh=mesh,
        out_type=jax.ShapeDtypeStruct((groups * _IDXW, _TW), F32),
        scratch_types=[
            pltpu.VMEM((per_w_groups, _IDXW), jnp.int32),
            pltpu.VMEM((2, _IDXW, _TW), F32),
            pltpu.VMEM_SHARED((nrows, _TW), F32),
            pltpu.SemaphoreType.DMA,
            pltpu.SemaphoreType.DMA,
            pltpu.SemaphoreType.DMA,
            pltpu.SemaphoreType.DMA,
        ],
        compiler_params=pltpu.CompilerParams(use_tc_tiling_on_sc=False),
    )
    def k(table_hbm, idx_hbm, out_hbm, idx_v, rows_v, tbl_sh,
          gs0, gs1, ss0, ss1):
        cid = lax.axis_index("c")
        sid = lax.axis_index("s")
        wid = sid * _NC + cid
        gbase = wid * per_w_groups

        pltpu.sync_copy(idx_hbm.at[pl.ds(gbase, per_w_groups)], idx_v)
        pltpu.sync_copy(table_hbm.at[pl.ds(sid * rows_per_sub, rows_per_sub)],
                        tbl_sh.at[pl.ds(sid * rows_per_sub, rows_per_sub)])
        plsc.subcore_barrier()

        gsems = (gs0, gs1)
        ssems = (ss0, ss1)

        def body(t, carry):
            c0 = 2 * t
            gh = []
            for bb in range(2):
                gh.append(pltpu.async_copy(
                    tbl_sh.at[idx_v.at[c0 + bb]], rows_v.at[bb], gsems[bb]))
            sh = []
            for bb in range(2):
                gh[bb].wait()
                sh.append(pltpu.async_copy(
                    rows_v.at[bb],
                    out_hbm.at[pl.ds((gbase + c0 + bb) * _IDXW, _IDXW)],
                    ssems[bb]))
            for bb in range(2):
                sh[bb].wait()
            return carry

        lax.fori_loop(0, iters, body, 0)

    return k(table, idx2)


def _mlp_table(xs, p3, wbs, wf, wq, consts, nb=1024):
    n = xs[0].shape[1]
    npc = len(xs)

    def body(*refs):
        x_refs = refs[:npc]
        p_ref = refs[npc]
        wb_refs = refs[npc + 1:2 * npc + 1]
        wf_ref, wq_ref, c_ref, o_ref = refs[2 * npc + 1:]
        acc = c_ref[:, 0:1]
        for xr, wr in zip(x_refs, wb_refs):
            acc = acc + jnp.dot(wr[...], xr[...], preferred_element_type=F32)
        new = jnp.maximum(acc, 0.0)
        f = jnp.maximum(
            jnp.dot(wf_ref[...], new, preferred_element_type=F32)
            + c_ref[:, 1:2], 0.0)
        q = jnp.dot(wq_ref[...], p_ref[...], preferred_element_type=F32)
        o_ref[...] = jnp.concatenate([f, q], axis=0).T

    in_specs = (
        [pl.BlockSpec((x.shape[0], nb), lambda i: (0, i)) for x in xs]
        + [pl.BlockSpec((3, nb), lambda i: (0, i))]
        + [pl.BlockSpec(w.shape, lambda i: (0, 0)) for w in wbs]
        + [pl.BlockSpec(wf.shape, lambda i: (0, 0)),
           pl.BlockSpec(wq.shape, lambda i: (0, 0)),
           pl.BlockSpec((64, 8), lambda i: (0, 0))]
    )
    return pl.pallas_call(
        body,
        grid=(-(-n // nb),),
        in_specs=in_specs,
        out_specs=pl.BlockSpec((nb, _TW), lambda i: (i, 0)),
        out_shape=jax.ShapeDtypeStruct((n, _TW), F32),
    )(*xs, p3, *wbs, wf, wq, consts)


def _edge(g3, table, wp, consts, n, kslc, acc_in=None, final=True, nb=512):
    def body(*refs):
        if acc_in is None:
            g_ref, t_ref, wp_ref, c_ref, o_ref = refs
            acc = jnp.zeros((nb, 64), F32)
        else:
            g_ref, t_ref, wp_ref, c_ref, a_ref, o_ref = refs
            acc = a_ref[...]
        lane = lax.broadcasted_iota(jnp.int32, (1, _TW), 1)
        qmask = jnp.where(lane >= 64, 1.0, 0.0)
        s = t_ref[...] * qmask - c_ref[0:1, :]
        cp = c_ref[1:2, :64]
        wp2 = wp_ref[...]
        for kk in range(kslc):
            u = jnp.maximum(g_ref[kk] - s, 0.0)
            w = u * pltpu.roll(u, 64, 1)
            acc = acc + jnp.maximum(
                jnp.dot(w, wp2, preferred_element_type=F32) + cp, 0.0)
        o_ref[...] = acc.T if final else acc

    in_specs = [
        pl.BlockSpec((kslc, nb, _TW), lambda i: (0, i, 0)),
        pl.BlockSpec((nb, _TW), lambda i: (i, 0)),
        pl.BlockSpec((_TW, 64), lambda i: (0, 0)),
        pl.BlockSpec((8, _TW), lambda i: (0, 0)),
    ]
    args = [g3, table, wp, consts]
    if acc_in is not None:
        in_specs.append(pl.BlockSpec((nb, 64), lambda i: (i, 0)))
        args.append(acc_in)
    if final:
        out_spec = pl.BlockSpec((64, nb), lambda i: (0, i))
        out_shape = jax.ShapeDtypeStruct((64, n), F32)
    else:
        out_spec = pl.BlockSpec((nb, 64), lambda i: (i, 0))
        out_shape = jax.ShapeDtypeStruct((n, 64), F32)
    return pl.pallas_call(
        body,
        grid=(-(-n // nb),),
        in_specs=in_specs,
        out_specs=out_spec,
        out_shape=out_shape,
    )(*args)


def _fold(w, b, g, bt):
    return w * g[:, None], b * g + bt


def _consts_rows(*rows):
    m = jnp.stack(rows)
    return jnp.pad(m, ((0, 8 - m.shape[0]), (0, 0)))


def kernel(feats, pts, knn_idx, params):
    b, fdim, n = feats.shape
    k = knn_idx.shape[2]
    e = n * k
    chunk = _NW * _CB * _IDXW
    epad = chunk * (-(-e // chunk))

    x0 = feats[0]
    p3 = pts[0]
    npad = epad // k
    idx_km = jnp.pad(knn_idx[0].T, ((0, 0), (0, npad - n)))
    idx2 = idx_km.reshape(epad // _IDXW, _IDXW)

    fold = []
    for p in params:
        wb, cb = _fold(p['Wb'], p['bb'], p['gb'], p['betab'])
        wf, cf = _fold(p['Wf'], p['bf'], p['gf'], p['betaf'])
        wq, cd = _fold(p['Wd'], p['bd'], p['gd'], p['betad'])
        wp, cp = _fold(p['Wp'], p['bp'], p['gp'], p['betap'])
        z64 = jnp.zeros((64,), F32)
        fold.append(dict(
            wb=wb, wf=wf, wq=wq,
            wp=jnp.pad(wp.T, ((0, 64), (0, 0))),
            cab=jnp.pad(jnp.stack([cb, cf], axis=1), ((0, 0), (0, 6))),
            ccd=_consts_rows(jnp.concatenate([z64, cd]),
                             jnp.concatenate([cp, z64]))))

    outs = []
    xpieces, wsplit = [x0], [fdim]
    for i, f in enumerate(fold):
        splits = []
        off = 0
        for w in wsplit:
            splits.append((off, w))
            off += w
        wbs = [f['wb'][:, o:o + w] for (o, w) in splits]
        table = _mlp_table(xpieces, p3, wbs, f['wf'], f['wq'], f['cab'])
        half = idx2.shape[0] // 2
        k2 = k // 2
        ga = _sc_gather(table, idx2[:half]).reshape(k2, npad, _TW)
        gb = _sc_gather(table, idx2[half:]).reshape(k2, npad, _TW)
        acc = _edge(ga, table, f['wp'], f['ccd'], n, k2, final=False)
        oi = _edge(gb, table, f['wp'], f['ccd'], n, k2, acc_in=acc)
        outs.append(oi)
        xpieces = xpieces + [oi]
        wsplit = wsplit + [64]

    return jnp.concatenate([feats] + [o[None] for o in outs], axis=1)


# speedup vs baseline: 1.3599x; 1.3599x over previous
import functools

import jax
import jax.numpy as jnp
from jax import lax
from jax.experimental import pallas as pl
from jax.experimental.pallas import tpu as pltpu
from jax.experimental.pallas import tpu_sc as plsc

F32 = jnp.float32
U32 = jnp.uint32

_NC, _NS = 2, 16
_NW = _NC * _NS
_IDXW = 128
_TW = 64
_CBUF = 4


def _sc_gather(table, idx2):
    groups = idx2.shape[0]
    nrows = table.shape[0]
    per_w_groups = groups // _NW
    iters = per_w_groups // (2 * _CBUF)
    rows_per_sub = nrows // _NS
    mesh = plsc.VectorSubcoreMesh(core_axis_name="c", subcore_axis_name="s")

    @functools.partial(
        pl.kernel,
        mesh=mesh,
        out_type=jax.ShapeDtypeStruct((groups * _IDXW, _TW), F32),
        scratch_types=[
            pltpu.VMEM((per_w_groups, _IDXW), jnp.int32),
            pltpu.VMEM((2, _CBUF * _IDXW, _TW), F32),
            pltpu.VMEM_SHARED((nrows, _TW), F32),
            pltpu.SemaphoreType.DMA,
            pltpu.SemaphoreType.DMA,
            pltpu.SemaphoreType.DMA,
            pltpu.SemaphoreType.DMA,
        ],
        compiler_params=pltpu.CompilerParams(use_tc_tiling_on_sc=False),
    )
    def k(table_hbm, idx_hbm, out_hbm, idx_v, rows_v, tbl_sh,
          gs0, gs1, ss0, ss1):
        cid = lax.axis_index("c")
        sid = lax.axis_index("s")
        wid = sid * _NC + cid
        gbase = wid * per_w_groups

        pltpu.sync_copy(idx_hbm.at[pl.ds(gbase, per_w_groups)], idx_v)
        pltpu.sync_copy(table_hbm.at[pl.ds(sid * rows_per_sub, rows_per_sub)],
                        tbl_sh.at[pl.ds(sid * rows_per_sub, rows_per_sub)])
        plsc.subcore_barrier()

        gsems = (gs0, gs1)
        ssems = (ss0, ss1)

        def body(t, carry):
            c0 = 2 * _CBUF * t
            gh = []
            for bb in range(2):
                for j in range(_CBUF):
                    gh.append(pltpu.async_copy(
                        tbl_sh.at[idx_v.at[c0 + bb * _CBUF + j]],
                        rows_v.at[bb].at[pl.ds(j * _IDXW, _IDXW)],
                        gsems[bb]))
            sh = []
            for bb in range(2):
                for j in range(_CBUF):
                    gh[bb * _CBUF + j].wait()
                sh.append(pltpu.async_copy(
                    rows_v.at[bb],
                    out_hbm.at[pl.ds((gbase + c0 + bb * _CBUF) * _IDXW,
                                     _CBUF * _IDXW)],
                    ssems[bb]))
            for bb in range(2):
                sh[bb].wait()
            return carry

        lax.fori_loop(0, iters, body, 0)

    return k(table, idx2)


def _pack_bf16_pair(low_f32, high_f32):
    lo = lax.bitcast_convert_type(low_f32.astype(jnp.bfloat16),
                                  jnp.uint16).astype(U32)
    hi = lax.bitcast_convert_type(high_f32.astype(jnp.bfloat16),
                                  jnp.uint16).astype(U32)
    return lax.bitcast_convert_type(lo | (hi << 16), F32)


def _unpack_low(word_f32):
    u = lax.bitcast_convert_type(word_f32, U32)
    return lax.bitcast_convert_type(u << 16, F32)


def _unpack_high(word_f32):
    u = lax.bitcast_convert_type(word_f32, U32)
    return lax.bitcast_convert_type(u & jnp.uint32(0xFFFF0000), F32)


def _mlp_table(xs, p3, wbs, wf, wq, consts, sel_e, sel_o, nb=1024):
    n = xs[0].shape[1]
    npc = len(xs)

    def body(*refs):
        x_refs = refs[:npc]
        p_ref = refs[npc]
        wb_refs = refs[npc + 1:2 * npc + 1]
        wf_ref, wq_ref, c_ref, se_ref, so_ref, o_ref = refs[2 * npc + 1:]
        acc = c_ref[:, 0:1]
        for xr, wr in zip(x_refs, wb_refs):
            acc = acc + jnp.dot(wr[...], xr[...], preferred_element_type=F32)
        new = jnp.maximum(acc, 0.0)
        f = jnp.maximum(
            jnp.dot(wf_ref[...], new, preferred_element_type=F32)
            + c_ref[:, 1:2], 0.0)
        q = jnp.dot(wq_ref[...], p_ref[...], preferred_element_type=F32)
        se = se_ref[...]
        so = so_ref[...]
        fe = jnp.dot(f, se, preferred_element_type=F32).T
        fo = jnp.dot(f, so, preferred_element_type=F32).T
        qe = jnp.dot(q, se, preferred_element_type=F32).T
        qo = jnp.dot(q, so, preferred_element_type=F32).T
        o_ref[...] = jnp.concatenate(
            [_pack_bf16_pair(fe, qe), _pack_bf16_pair(fo, qo)], axis=1)

    in_specs = (
        [pl.BlockSpec((x.shape[0], nb), lambda i: (0, i)) for x in xs]
        + [pl.BlockSpec((3, nb), lambda i: (0, i))]
        + [pl.BlockSpec(w.shape, lambda i: (0, 0)) for w in wbs]
        + [pl.BlockSpec(wf.shape, lambda i: (0, 0)),
           pl.BlockSpec(wq.shape, lambda i: (0, 0)),
           pl.BlockSpec((64, 8), lambda i: (0, 0)),
           pl.BlockSpec(sel_e.shape, lambda i: (0, 0)),
           pl.BlockSpec(sel_o.shape, lambda i: (0, 0))]
    )
    return pl.pallas_call(
        body,
        grid=(n // nb,),
        in_specs=in_specs,
        out_specs=pl.BlockSpec((nb // 2, 128), lambda i: (i, 0)),
        out_shape=jax.ShapeDtypeStruct((n // 2, 128), F32),
    )(*xs, p3, *wbs, wf, wq, consts, sel_e, sel_o)


def _edge(g3, tablev, wp2, consts, n, kslc, inter=None, acc_in=None,
          final=True, nb=512):
    nb2 = nb // 2

    def body(*refs):
        g_ref, t_ref, wp_ref, c_ref = refs[:4]
        if acc_in is None:
            acc = jnp.zeros((nb2, 128), F32)
        else:
            acc = refs[4][...]
        o_ref = refs[-1]
        qs = _unpack_high(t_ref[...])
        s = qs - c_ref[0:1, :]
        cp = c_ref[1:2, :]
        wpm = wp_ref[...]
        for kk in range(kslc):
            blk = g_ref[kk]
            g = _unpack_low(blk)
            qn = _unpack_high(blk)
            d = jnp.maximum(qn - s, 0.0)
            h = d * g
            acc = acc + jnp.maximum(
                jnp.dot(h, wpm, preferred_element_type=F32) + cp, 0.0)
        if final:
            at = acc.T
            ie_ref, io_ref = refs[-3], refs[-2]
            o_ref[...] = (
                jnp.dot(at[0:64], ie_ref[...], preferred_element_type=F32)
                + jnp.dot(at[64:128], io_ref[...], preferred_element_type=F32))
        else:
            o_ref[...] = acc

    in_specs = [
        pl.BlockSpec((kslc, nb2, 128), lambda i: (0, i, 0)),
        pl.BlockSpec((nb2, 128), lambda i: (i, 0)),
        pl.BlockSpec((128, 128), lambda i: (0, 0)),
        pl.BlockSpec((8, 128), lambda i: (0, 0)),
    ]
    args = [g3, tablev, wp2, consts]
    if acc_in is not None:
        in_specs.append(pl.BlockSpec((nb2, 128), lambda i: (i, 0)))
        args.append(acc_in)
    if final:
        ie, io = inter
        in_specs.append(pl.BlockSpec(ie.shape, lambda i: (0, 0)))
        in_specs.append(pl.BlockSpec(io.shape, lambda i: (0, 0)))
        args.append(ie)
        args.append(io)
        out_spec = pl.BlockSpec((64, nb), lambda i: (0, i))
        out_shape = jax.ShapeDtypeStruct((64, n), F32)
    else:
        out_spec = pl.BlockSpec((nb2, 128), lambda i: (i, 0))
        out_shape = jax.ShapeDtypeStruct((n // 2, 128), F32)
    return pl.pallas_call(
        body,
        grid=(n // nb,),
        in_specs=in_specs,
        out_specs=out_spec,
        out_shape=out_shape,
    )(*args)


def _fold(w, b, g, bt):
    return w * g[:, None], b * g + bt


def _consts_rows(*rows):
    m = jnp.stack(rows)
    return jnp.pad(m, ((0, 8 - m.shape[0]), (0, 0)))


def kernel(feats, pts, knn_idx, params):
    b, fdim, n = feats.shape
    k = knn_idx.shape[2]
    e = n * k
    chunk = _NW * 2 * _CBUF * _IDXW
    epad = chunk * (-(-e // chunk))

    npad = epad // k
    x0 = jnp.pad(feats[0], ((0, 0), (0, npad - n)))
    p3 = jnp.pad(pts[0], ((0, 0), (0, npad - n)))
    idx_km = jnp.pad(knn_idx[0].T, ((0, 0), (0, npad - n)))
    idx2 = idx_km.reshape(epad // _IDXW, _IDXW)

    fold = []
    for p in params:
        wb, cb = _fold(p['Wb'], p['bb'], p['gb'], p['betab'])
        wf, cf = _fold(p['Wf'], p['bf'], p['gf'], p['betaf'])
        wq, cd = _fold(p['Wd'], p['bd'], p['gd'], p['betad'])
        wp, cp = _fold(p['Wp'], p['bp'], p['gp'], p['betap'])
        z = jnp.zeros((64, 64), F32)
        wp2 = jnp.concatenate([
            jnp.concatenate([wp.T, z], axis=1),
            jnp.concatenate([z, wp.T], axis=1)], axis=0)
        fold.append(dict(
            wb=wb, wf=wf, wq=wq, wp2=wp2,
            cab=jnp.pad(jnp.stack([cb, cf], axis=1), ((0, 0), (0, 6))),
            ccd=_consts_rows(jnp.tile(cd, 2), jnp.tile(cp, 2))))

    outs = []
    xpieces, wsplit = [x0], [fdim]
    half = idx2.shape[0] // 2
    k2 = k // 2
    nbm, nbe = 1024, 512
    am = jnp.arange(nbm)[:, None]
    mm = jnp.arange(nbm // 2)[None, :]
    sel_e = (am == 2 * mm).astype(F32)
    sel_o = (am == 2 * mm + 1).astype(F32)
    me = jnp.arange(nbe // 2)[:, None]
    ae = jnp.arange(nbe)[None, :]
    inter_e = (2 * me == ae).astype(F32)
    inter_o = (2 * me + 1 == ae).astype(F32)
    for i, f in enumerate(fold):
        splits = []
        off = 0
        for w in wsplit:
            splits.append((off, w))
            off += w
        wbs = [f['wb'][:, o:o + w] for (o, w) in splits]
        tablev = _mlp_table(xpieces, p3, wbs, f['wf'], f['wq'], f['cab'],
                            sel_e, sel_o, nb=nbm)
        table = tablev.reshape(npad, _TW)
        ga = _sc_gather(table, idx2[:half]).reshape(k2, npad // 2, 128)
        gb = _sc_gather(table, idx2[half:]).reshape(k2, npad // 2, 128)
        acc = _edge(ga, tablev, f['wp2'], f['ccd'], npad, k2, final=False,
                    nb=nbe)
        oi = _edge(gb, tablev, f['wp2'], f['ccd'], npad, k2,
                   inter=(inter_e, inter_o), acc_in=acc, nb=nbe)
        outs.append(oi)
        xpieces = xpieces + [oi]
        wsplit = wsplit + [64]

    return jnp.concatenate([feats] + [o[None, :, :n] for o in outs], axis=1)
